# Initial kernel scaffold; baseline (speedup 1.0000x reference)
import jax
import jax.numpy as jnp
from jax.experimental import pallas as pl


def kernel(manufacturing_costs, conservation, sequence, variant_profiles):
    raise NotImplementedError("write your pallas kernel here")


# speedup vs baseline: 1.7489x; 1.7489x over previous
import functools

import jax
import jax.numpy as jnp
from jax import lax
from jax.experimental import pallas as pl
from jax.experimental.pallas import tpu as pltpu
from jax.experimental.pallas import tpu_sc as plsc

_N_SITES = 2048
_N_STATES = 21
_N_VARIANTS = 10
_BATCH = 1024
_ESCAPE_W, _MANU_W, _AUTO_W, _BREADTH_W = 1.0, 0.3, 0.5, 0.4
_GL_CENTER, _GL_WIDTH = 0.5, 0.15

_L = 16
_info = plsc.get_sparse_core_info()
_NC, _NS = _info.num_cores, _info.num_subcores
_NW = _NC * _NS
_ROWS_PER_W = _BATCH // _NW
_CHUNKS = _N_SITES // _L
_COSTS_PAD = 32


def _body(costs_hbm, cons_hbm, seq_hbm, var_hbm, out_hbm,
          costs_v, cons_v, var_v, rows_v, out_v, sem_a, sem_b):
    wid = lax.axis_index("s") * _NC + lax.axis_index("c")
    base = wid * _ROWS_PER_W

    pltpu.sync_copy(costs_hbm, costs_v)
    pltpu.sync_copy(cons_hbm, cons_v)
    pltpu.sync_copy(var_hbm, var_v)

    def cons_body(i, acc):
        return acc + cons_v[pl.ds(i * _L, _L)]

    cacc = lax.fori_loop(0, _CHUNKS, cons_body, jnp.zeros((_L,), jnp.float32))
    cons_mean = jnp.sum(cacc) * jnp.float32(1.0 / _N_SITES)
    z = (cons_mean * jnp.float32(1.0 / _N_STATES) - jnp.float32(_GL_CENTER)) \
        * jnp.float32(1.0 / _GL_WIDTH)
    gl_vec = jnp.exp(jnp.full((_L,), jnp.float32(-0.5)) * z * z)
    e_auto = jnp.float32(1.0) - jnp.sum(gl_vec) * jnp.float32(1.0 / _L)

    sems = (sem_a, sem_b)
    copies = [None, None]
    copies[0] = pltpu.async_copy(seq_hbm.at[base], rows_v.at[0], sems[0])

    zero_i = jnp.zeros((_L,), jnp.int32)
    zero_f = jnp.zeros((_L,), jnp.float32)
    lane_iota = lax.iota(jnp.int32, _L)
    out_acc = zero_f

    for r in range(_ROWS_PER_W):
        cur = r % 2
        if r + 1 < _ROWS_PER_W:
            nxt = (r + 1) % 2
            copies[nxt] = pltpu.async_copy(
                seq_hbm.at[base + (r + 1)], rows_v.at[nxt], sems[nxt])
        copies[cur].wait()

        def chunk_body(c, carry, cur=cur):
            cost_acc = carry[0]
            cnts = carry[1:]
            sv = rows_v[cur, pl.ds(c * _L, _L)]
            cost_acc = cost_acc + plsc.load_gather(costs_v, [sv])
            new = []
            for v in range(_N_VARIANTS):
                eq = sv == var_v[v, pl.ds(c * _L, _L)]
                new.append(cnts[v] + eq.astype(jnp.int32))
            return (cost_acc,) + tuple(new)

        init = (zero_f,) + (zero_i,) * _N_VARIANTS
        res = lax.fori_loop(0, _CHUNKS, chunk_body, init)
        cost_sum = jnp.sum(res[0])
        cnt = [jnp.sum(res[1 + v]) for v in range(_N_VARIANTS)]

        cov_total = cnt[0]
        covered = (cnt[0] > _N_SITES // 2).astype(jnp.float32)
        for v in range(1, _N_VARIANTS):
            cov_total = cov_total + cnt[v]
            covered = covered + (cnt[v] > _N_SITES // 2).astype(jnp.float32)

        e_escape = jnp.float32(1.0) - cov_total.astype(jnp.float32) * \
            jnp.float32(1.0 / (_N_VARIANTS * _N_SITES))
        e_manu = cost_sum * jnp.float32(1.0 / _N_SITES)
        e_breadth = jnp.float32(1.0) - covered * jnp.float32(1.0 / _N_VARIANTS)
        total_r = (jnp.float32(_ESCAPE_W) * e_escape
                   + jnp.float32(_MANU_W) * e_manu
                   + jnp.float32(_AUTO_W) * e_auto
                   + jnp.float32(_BREADTH_W) * e_breadth)
        out_acc = jnp.where(lane_iota == (r % _L),
                            jnp.full((_L,), total_r), out_acc)
        if r % _L == _L - 1:
            out_v[pl.ds((r // _L) * _L, _L)] = out_acc
            out_acc = zero_f

    pltpu.sync_copy(out_v, out_hbm.at[pl.ds(base, _ROWS_PER_W)])


@functools.partial(
    pl.kernel,
    mesh=plsc.VectorSubcoreMesh(core_axis_name="c", subcore_axis_name="s"),
    out_type=jax.ShapeDtypeStruct((_BATCH,), jnp.float32),
    scratch_types=[
        pltpu.VMEM((_COSTS_PAD,), jnp.float32),
        pltpu.VMEM((_N_SITES,), jnp.float32),
        pltpu.VMEM((_N_VARIANTS, _N_SITES), jnp.int32),
        pltpu.VMEM((2, _N_SITES), jnp.int32),
        pltpu.VMEM((_ROWS_PER_W,), jnp.float32),
        pltpu.SemaphoreType.DMA,
        pltpu.SemaphoreType.DMA,
    ],
    compiler_params=pltpu.CompilerParams(needs_layout_passes=False),
)
def _sc_landscape(*refs):
    _body(*refs)


def kernel(manufacturing_costs, conservation, sequence, variant_profiles):
    costs_pad = jnp.zeros((_COSTS_PAD,), jnp.float32)
    costs_pad = costs_pad.at[:_N_STATES].set(
        manufacturing_costs.astype(jnp.float32))
    return _sc_landscape(costs_pad, conservation.astype(jnp.float32),
                         sequence, variant_profiles)


# speedup vs baseline: 2.2410x; 1.2814x over previous
import functools

import jax
import jax.numpy as jnp
from jax import lax
from jax.experimental import pallas as pl
from jax.experimental.pallas import tpu as pltpu
from jax.experimental.pallas import tpu_sc as plsc

_N_SITES = 2048
_N_STATES = 21
_N_VARIANTS = 10
_BATCH = 1024
_ESCAPE_W, _MANU_W, _AUTO_W, _BREADTH_W = 1.0, 0.3, 0.5, 0.4
_GL_CENTER, _GL_WIDTH = 0.5, 0.15

_L = 16
_info = plsc.get_sparse_core_info()
_NC, _NS = _info.num_cores, _info.num_subcores
_NW = _NC * _NS
_ROWS_PER_W = _BATCH // _NW
_CHUNKS = _N_SITES // _L
_COSTS_PAD = 32

_TC_BLK = 128


def _sc_body(costs_hbm, cons_hbm, seq_hbm, out_hbm,
             costs_v, cons_v, rows_v, out_v, sem_a, sem_b):
    wid = lax.axis_index("s") * _NC + lax.axis_index("c")
    base = wid * _ROWS_PER_W

    pltpu.sync_copy(costs_hbm, costs_v)
    pltpu.sync_copy(cons_hbm, cons_v)

    def cons_body(i, acc):
        return acc + cons_v[pl.ds(i * _L, _L)]

    cacc = lax.fori_loop(0, _CHUNKS, cons_body, jnp.zeros((_L,), jnp.float32))
    cons_mean = jnp.sum(cacc) * jnp.float32(1.0 / _N_SITES)
    z = (cons_mean * jnp.float32(1.0 / _N_STATES) - jnp.float32(_GL_CENTER)) \
        * jnp.float32(1.0 / _GL_WIDTH)
    gl_vec = jnp.exp(jnp.full((_L,), jnp.float32(-0.5)) * z * z)
    e_auto = jnp.float32(1.0) - jnp.sum(gl_vec) * jnp.float32(1.0 / _L)

    sems = (sem_a, sem_b)
    copies = [None, None]
    copies[0] = pltpu.async_copy(seq_hbm.at[base], rows_v.at[0], sems[0])

    zero_f = jnp.zeros((_L,), jnp.float32)
    lane_iota = lax.iota(jnp.int32, _L)
    out_acc = zero_f

    for r in range(_ROWS_PER_W):
        cur = r % 2
        if r + 1 < _ROWS_PER_W:
            nxt = (r + 1) % 2
            copies[nxt] = pltpu.async_copy(
                seq_hbm.at[base + (r + 1)], rows_v.at[nxt], sems[nxt])
        copies[cur].wait()

        def chunk_body(c, cost_acc, cur=cur):
            sv = rows_v[cur, pl.ds(c * _L, _L)]
            return cost_acc + plsc.load_gather(costs_v, [sv])

        cost_vec = lax.fori_loop(0, _CHUNKS, chunk_body, zero_f)
        cost_sum = jnp.sum(cost_vec)

        total_r = (jnp.float32(_MANU_W / _N_SITES) * cost_sum
                   + jnp.float32(_AUTO_W) * e_auto)
        out_acc = jnp.where(lane_iota == (r % _L),
                            jnp.full((_L,), total_r), out_acc)
        if r % _L == _L - 1:
            out_v[pl.ds((r // _L) * _L, _L)] = out_acc
            out_acc = zero_f

    pltpu.sync_copy(out_v, out_hbm.at[pl.ds(base, _ROWS_PER_W)])


@functools.partial(
    pl.kernel,
    mesh=plsc.VectorSubcoreMesh(core_axis_name="c", subcore_axis_name="s"),
    out_type=jax.ShapeDtypeStruct((_BATCH,), jnp.float32),
    scratch_types=[
        pltpu.VMEM((_COSTS_PAD,), jnp.float32),
        pltpu.VMEM((_N_SITES,), jnp.float32),
        pltpu.VMEM((2, _N_SITES), jnp.int32),
        pltpu.VMEM((_ROWS_PER_W,), jnp.float32),
        pltpu.SemaphoreType.DMA,
        pltpu.SemaphoreType.DMA,
    ],
    compiler_params=pltpu.CompilerParams(needs_layout_passes=False),
)
def _sc_costs(*refs):
    _sc_body(*refs)


def _tc_body(seq_ref, var_ref, out_ref):
    seq = seq_ref[...]
    cov = jnp.zeros((_TC_BLK,), jnp.int32)
    covered = jnp.zeros((_TC_BLK,), jnp.int32)
    for v in range(_N_VARIANTS):
        eq = (seq == var_ref[v][None, :]).astype(jnp.int32)
        cnt = jnp.sum(eq, axis=1)
        cov = cov + cnt
        covered = covered + (cnt > _N_SITES // 2).astype(jnp.int32)
    e_escape = jnp.float32(1.0) - cov.astype(jnp.float32) * \
        jnp.float32(1.0 / (_N_VARIANTS * _N_SITES))
    e_breadth = jnp.float32(1.0) - covered.astype(jnp.float32) * \
        jnp.float32(1.0 / _N_VARIANTS)
    out_ref[...] = (jnp.float32(_ESCAPE_W) * e_escape
                    + jnp.float32(_BREADTH_W) * e_breadth)


_tc_matches = pl.pallas_call(
    _tc_body,
    grid=(_BATCH // _TC_BLK,),
    in_specs=[
        pl.BlockSpec((_TC_BLK, _N_SITES), lambda i: (i, 0)),
        pl.BlockSpec((_N_VARIANTS, _N_SITES), lambda i: (0, 0)),
    ],
    out_specs=pl.BlockSpec((_TC_BLK,), lambda i: (i,)),
    out_shape=jax.ShapeDtypeStruct((_BATCH,), jnp.float32),
    compiler_params=pltpu.CompilerParams(
        dimension_semantics=("arbitrary",)),
)


def kernel(manufacturing_costs, conservation, sequence, variant_profiles):
    costs_pad = jnp.zeros((_COSTS_PAD,), jnp.float32)
    costs_pad = costs_pad.at[:_N_STATES].set(
        manufacturing_costs.astype(jnp.float32))
    sc_part = _sc_costs(costs_pad, conservation.astype(jnp.float32), sequence)
    tc_part = _tc_matches(sequence, variant_profiles)
    return sc_part + tc_part


# speedup vs baseline: 2.6577x; 1.1860x over previous
import functools

import jax
import jax.numpy as jnp
from jax import lax
from jax.experimental import pallas as pl
from jax.experimental.pallas import tpu as pltpu
from jax.experimental.pallas import tpu_sc as plsc

_N_SITES = 2048
_N_STATES = 21
_N_VARIANTS = 10
_BATCH = 1024
_ESCAPE_W, _MANU_W, _AUTO_W, _BREADTH_W = 1.0, 0.3, 0.5, 0.4
_GL_CENTER, _GL_WIDTH = 0.5, 0.15

_L = 16
_info = plsc.get_sparse_core_info()
_NC, _NS = _info.num_cores, _info.num_subcores
_NW = _NC * _NS
_ROWS_PER_W = _BATCH // _NW
_HALF = _ROWS_PER_W // 2
_PACK = 4
_WORDS = _N_SITES // _PACK
_WREGS = _WORDS // _L
_COSTS_PAD = 32

_TC_BLK = 128


def _sc_body(costs_hbm, cons_hbm, packed_hbm, out_hbm,
             costs_v, cons_v, rows_v, out_v, sem_a, sem_b, sem_c):
    wid = lax.axis_index("s") * _NC + lax.axis_index("c")
    base = wid * _ROWS_PER_W

    pltpu.sync_copy(costs_hbm, costs_v)
    cons_cp = pltpu.async_copy(cons_hbm, cons_v, sem_c)
    copies = [
        pltpu.async_copy(packed_hbm.at[pl.ds(base, _HALF)],
                         rows_v.at[0], sem_a),
        pltpu.async_copy(packed_hbm.at[pl.ds(base + _HALF, _HALF)],
                         rows_v.at[1], sem_b),
    ]

    cons_cp.wait()

    def cons_body(i, acc):
        return acc + cons_v[pl.ds(i * _L, _L)]

    cacc = lax.fori_loop(0, _N_SITES // _L, cons_body,
                         jnp.zeros((_L,), jnp.float32), unroll=4)
    cons_mean = jnp.sum(cacc) * jnp.float32(1.0 / _N_SITES)
    z = (cons_mean * jnp.float32(1.0 / _N_STATES) - jnp.float32(_GL_CENTER)) \
        * jnp.float32(1.0 / _GL_WIDTH)
    gl_vec = jnp.exp(jnp.full((_L,), jnp.float32(-0.5)) * z * z)
    e_auto = jnp.float32(1.0) - jnp.sum(gl_vec) * jnp.float32(1.0 / _L)
    base_e = jnp.float32(_AUTO_W) * e_auto

    lane_iota = lax.iota(jnp.int32, _L)
    mask5 = jnp.full((_L,), 0x1F, jnp.int32)

    for h in range(2):
        copies[h].wait()

        def row_body(r, out_acc, h=h):
            def w_body(i, cost_vec):
                cv = cost_vec
                for k4 in range(4):
                    w = rows_v[h, r, pl.ds((i * 4 + k4) * _L, _L)]
                    for k in range(_PACK):
                        sv = (w >> (8 * k)) & mask5 if k else w & mask5
                        cv = cv + plsc.load_gather(costs_v, [sv])
                return cv

            cost_vec = lax.fori_loop(0, _WREGS // 4, w_body,
                                     jnp.zeros((_L,), jnp.float32))
            total_r = jnp.float32(_MANU_W / _N_SITES) * jnp.sum(cost_vec) \
                + base_e
            return jnp.where(lane_iota == r, jnp.full((_L,), total_r),
                             out_acc)

        out_acc = lax.fori_loop(0, _HALF, row_body,
                                jnp.zeros((_L,), jnp.float32))
        out_v[pl.ds(h * _HALF, _HALF)] = out_acc

    pltpu.sync_copy(out_v, out_hbm.at[pl.ds(base, _ROWS_PER_W)])


@functools.partial(
    pl.kernel,
    mesh=plsc.VectorSubcoreMesh(core_axis_name="c", subcore_axis_name="s"),
    out_type=jax.ShapeDtypeStruct((_BATCH,), jnp.float32),
    scratch_types=[
        pltpu.VMEM((_COSTS_PAD,), jnp.float32),
        pltpu.VMEM((_N_SITES,), jnp.float32),
        pltpu.VMEM((2, _HALF, _WORDS), jnp.int32),
        pltpu.VMEM((_ROWS_PER_W,), jnp.float32),
        pltpu.SemaphoreType.DMA,
        pltpu.SemaphoreType.DMA,
        pltpu.SemaphoreType.DMA,
    ],
    compiler_params=pltpu.CompilerParams(needs_layout_passes=False),
)
def _sc_costs(*refs):
    _sc_body(*refs)


def _tc_body(seq_ref, var_ref, out_ref, packed_ref):
    seq = seq_ref[...]
    cov = jnp.zeros((_TC_BLK,), jnp.int32)
    covered = jnp.zeros((_TC_BLK,), jnp.int32)
    for v in range(_N_VARIANTS):
        cnt = jnp.count_nonzero(seq == var_ref[v][None, :], axis=1)
        cov = cov + cnt
        covered = covered + (cnt > _N_SITES // 2).astype(jnp.int32)
    e_escape = jnp.float32(1.0) - cov.astype(jnp.float32) * \
        jnp.float32(1.0 / (_N_VARIANTS * _N_SITES))
    e_breadth = jnp.float32(1.0) - covered.astype(jnp.float32) * \
        jnp.float32(1.0 / _N_VARIANTS)
    out_ref[...] = (jnp.float32(_ESCAPE_W) * e_escape
                    + jnp.float32(_BREADTH_W) * e_breadth)
    p = seq[:, 0:_WORDS]
    for k in range(1, _PACK):
        p = p | (seq[:, k * _WORDS:(k + 1) * _WORDS] << (8 * k))
    packed_ref[...] = p


_tc_matches = pl.pallas_call(
    _tc_body,
    grid=(_BATCH // _TC_BLK,),
    in_specs=[
        pl.BlockSpec((_TC_BLK, _N_SITES), lambda i: (i, 0)),
        pl.BlockSpec((_N_VARIANTS, _N_SITES), lambda i: (0, 0)),
    ],
    out_specs=[
        pl.BlockSpec((_TC_BLK,), lambda i: (i,)),
        pl.BlockSpec((_TC_BLK, _WORDS), lambda i: (i, 0)),
    ],
    out_shape=[
        jax.ShapeDtypeStruct((_BATCH,), jnp.float32),
        jax.ShapeDtypeStruct((_BATCH, _WORDS), jnp.int32),
    ],
    compiler_params=pltpu.CompilerParams(
        dimension_semantics=("arbitrary",)),
)


def kernel(manufacturing_costs, conservation, sequence, variant_profiles):
    costs_pad = jnp.zeros((_COSTS_PAD,), jnp.float32)
    costs_pad = costs_pad.at[:_N_STATES].set(
        manufacturing_costs.astype(jnp.float32))
    tc_part, packed = _tc_matches(sequence, variant_profiles)
    sc_part = _sc_costs(costs_pad, conservation.astype(jnp.float32), packed)
    return sc_part + tc_part


# speedup vs baseline: 2.8529x; 1.0734x over previous
import functools

import jax
import jax.numpy as jnp
from jax import lax
from jax.experimental import pallas as pl
from jax.experimental.pallas import tpu as pltpu
from jax.experimental.pallas import tpu_sc as plsc

_N_SITES = 2048
_N_STATES = 21
_N_VARIANTS = 10
_BATCH = 1024
_ESCAPE_W, _MANU_W, _AUTO_W, _BREADTH_W = 1.0, 0.3, 0.5, 0.4
_GL_CENTER, _GL_WIDTH = 0.5, 0.15

_L = 16
_info = plsc.get_sparse_core_info()
_NC, _NS = _info.num_cores, _info.num_subcores
_NW = _NC * _NS
_ROWS_PER_W = _BATCH // _NW
_QUART = _ROWS_PER_W // 4
_PACK = 4
_WORDS = _N_SITES // _PACK
_WREGS = _WORDS // _L
_COSTS_PAD = 32

_TC_BLK = 128


def _sc_body(costs_hbm, packed_hbm, tc_hbm, out_hbm,
             costs_v, rows_v, tc_v, out_v,
             sem_a, sem_b, sem_c, sem_d, sem_e):
    wid = lax.axis_index("s") * _NC + lax.axis_index("c")
    base = wid * _ROWS_PER_W

    pltpu.sync_copy(costs_hbm, costs_v)
    tc_cp = pltpu.async_copy(tc_hbm.at[pl.ds(base, _ROWS_PER_W)],
                             tc_v, sem_e)
    sems = (sem_a, sem_b, sem_c, sem_d)
    copies = [
        pltpu.async_copy(packed_hbm.at[pl.ds(base + q * _QUART, _QUART)],
                         rows_v.at[q], sems[q])
        for q in range(4)
    ]

    lane_iota = lax.iota(jnp.int32, _L)
    mask5 = jnp.full((_L,), 0x1F, jnp.int32)
    tc_cp.wait()

    out_acc = jnp.zeros((_L,), jnp.float32)
    for q in range(4):
        copies[q].wait()

        def row_body(r, out_acc, q=q):
            def w_body(i, cost_vec):
                cv = cost_vec
                for k4 in range(4):
                    w = rows_v[q, r, pl.ds((i * 4 + k4) * _L, _L)]
                    for k in range(_PACK):
                        sv = (w >> (8 * k)) & mask5 if k else w & mask5
                        cv = cv + plsc.load_gather(costs_v, [sv])
                return cv

            cost_vec = lax.fori_loop(0, _WREGS // 4, w_body,
                                     jnp.zeros((_L,), jnp.float32))
            total_r = jnp.float32(_MANU_W / _N_SITES) * jnp.sum(cost_vec)
            return jnp.where(lane_iota == (q % 2) * _QUART + r,
                             jnp.full((_L,), total_r), out_acc)

        out_acc = lax.fori_loop(0, _QUART, row_body, out_acc)
        if q % 2 == 1:
            out_v[pl.ds((q // 2) * _L, _L)] = \
                out_acc + tc_v[pl.ds((q // 2) * _L, _L)]
            out_acc = jnp.zeros((_L,), jnp.float32)

    pltpu.sync_copy(out_v, out_hbm.at[pl.ds(base, _ROWS_PER_W)])


@functools.partial(
    pl.kernel,
    mesh=plsc.VectorSubcoreMesh(core_axis_name="c", subcore_axis_name="s"),
    out_type=jax.ShapeDtypeStruct((_BATCH,), jnp.float32),
    scratch_types=[
        pltpu.VMEM((_COSTS_PAD,), jnp.float32),
        pltpu.VMEM((4, _QUART, _WORDS), jnp.int32),
        pltpu.VMEM((_ROWS_PER_W,), jnp.float32),
        pltpu.VMEM((_ROWS_PER_W,), jnp.float32),
        pltpu.SemaphoreType.DMA,
        pltpu.SemaphoreType.DMA,
        pltpu.SemaphoreType.DMA,
        pltpu.SemaphoreType.DMA,
        pltpu.SemaphoreType.DMA,
    ],
    compiler_params=pltpu.CompilerParams(needs_layout_passes=False),
)
def _sc_costs(*refs):
    _sc_body(*refs)


def _tc_body(seq_ref, var_ref, cons_ref, out_ref, packed_ref):
    seq = seq_ref[...]
    p = seq[:, 0:_WORDS]
    for k in range(1, _PACK):
        p = p | (seq[:, k * _WORDS:(k + 1) * _WORDS] << (8 * k))
    packed_ref[...] = p

    var = var_ref[...]
    pv = var[:, 0:_WORDS]
    for k in range(1, _PACK):
        pv = pv | (var[:, k * _WORDS:(k + 1) * _WORDS] << (8 * k))

    hi = jnp.int32(-2139062144)
    cov = jnp.zeros((_TC_BLK,), jnp.int32)
    covered = jnp.zeros((_TC_BLK,), jnp.int32)
    for v in range(_N_VARIANTS):
        x = p ^ pv[v][None, :]
        t = (hi - x) & hi
        bytes4 = lax.shift_right_logical(
            t, jnp.full(t.shape, 7, jnp.int32))
        t2 = bytes4 + (bytes4 >> 16)
        t3 = (t2 + (t2 >> 8)) & 0xFF
        cnt = jnp.sum(t3, axis=1)
        cov = cov + cnt
        covered = covered + (cnt > _N_SITES // 2).astype(jnp.int32)
    e_escape = jnp.float32(1.0) - cov.astype(jnp.float32) * \
        jnp.float32(1.0 / (_N_VARIANTS * _N_SITES))
    e_breadth = jnp.float32(1.0) - covered.astype(jnp.float32) * \
        jnp.float32(1.0 / _N_VARIANTS)

    cons_mean = jnp.sum(cons_ref[...]) * jnp.float32(1.0 / _N_SITES)
    z = (cons_mean * jnp.float32(1.0 / _N_STATES) - jnp.float32(_GL_CENTER)) \
        * jnp.float32(1.0 / _GL_WIDTH)
    e_auto = jnp.float32(1.0) - jnp.exp(jnp.float32(-0.5) * z * z)

    out_ref[...] = (jnp.float32(_ESCAPE_W) * e_escape
                    + jnp.float32(_BREADTH_W) * e_breadth
                    + jnp.float32(_AUTO_W) * e_auto)


_tc_matches = pl.pallas_call(
    _tc_body,
    grid=(_BATCH // _TC_BLK,),
    in_specs=[
        pl.BlockSpec((_TC_BLK, _N_SITES), lambda i: (i, 0)),
        pl.BlockSpec((_N_VARIANTS, _N_SITES), lambda i: (0, 0)),
        pl.BlockSpec((1, _N_SITES), lambda i: (0, 0)),
    ],
    out_specs=[
        pl.BlockSpec((_TC_BLK,), lambda i: (i,)),
        pl.BlockSpec((_TC_BLK, _WORDS), lambda i: (i, 0)),
    ],
    out_shape=[
        jax.ShapeDtypeStruct((_BATCH,), jnp.float32),
        jax.ShapeDtypeStruct((_BATCH, _WORDS), jnp.int32),
    ],
    compiler_params=pltpu.CompilerParams(
        dimension_semantics=("arbitrary",)),
)


def kernel(manufacturing_costs, conservation, sequence, variant_profiles):
    costs_pad = jnp.zeros((_COSTS_PAD,), jnp.float32)
    costs_pad = costs_pad.at[:_N_STATES].set(
        manufacturing_costs.astype(jnp.float32))
    tc_part, packed = _tc_matches(
        sequence, variant_profiles,
        conservation.astype(jnp.float32).reshape(1, _N_SITES))
    return _sc_costs(costs_pad, packed, tc_part)


# speedup vs baseline: 2.9376x; 1.0297x over previous
import functools

import jax
import jax.numpy as jnp
from jax import lax
from jax.experimental import pallas as pl
from jax.experimental.pallas import tpu as pltpu
from jax.experimental.pallas import tpu_sc as plsc

_N_SITES = 2048
_N_STATES = 21
_N_VARIANTS = 10
_BATCH = 1024
_ESCAPE_W, _MANU_W, _AUTO_W, _BREADTH_W = 1.0, 0.3, 0.5, 0.4
_GL_CENTER, _GL_WIDTH = 0.5, 0.15

_L = 16
_info = plsc.get_sparse_core_info()
_NC, _NS = _info.num_cores, _info.num_subcores
_NW = _NC * _NS
_ROWS_PER_W = _BATCH // _NW
_QUART = _ROWS_PER_W // 4
_PACK = 4
_WORDS = _N_SITES // _PACK
_PAIRS = 6
_CWORDS = _N_SITES // _PAIRS + 1
_CVREGS = 22
_CPAD = _CVREGS * _L
_TBL = 1024
_COSTS_PAD = 32

_TC_BLK = 128


def _sc_body(costs_hbm, packed_hbm, tc_hbm, out_hbm,
             costs_v, tbl_v, rows_v, tc_v, out_v,
             sem_a, sem_b, sem_c, sem_d, sem_e):
    wid = lax.axis_index("s") * _NC + lax.axis_index("c")
    base = wid * _ROWS_PER_W

    pltpu.sync_copy(costs_hbm, costs_v)
    tc_cp = pltpu.async_copy(tc_hbm.at[pl.ds(base, _ROWS_PER_W)],
                             tc_v, sem_e)
    sems = (sem_a, sem_b, sem_c, sem_d)
    copies = [
        pltpu.async_copy(packed_hbm.at[pl.ds(base + q * _QUART, _QUART)],
                         rows_v.at[q], sems[q])
        for q in range(4)
    ]

    lane_iota = lax.iota(jnp.int32, _L)
    mask10 = jnp.full((_L,), 0x3FF, jnp.int32)
    mask5 = jnp.full((_L,), 0x1F, jnp.int32)

    def tbl_body(i, carry):
        idx = lane_iota + i * _L
        ga = plsc.load_gather(costs_v, [idx & mask5])
        gb = plsc.load_gather(costs_v, [idx >> 5])
        tbl_v[pl.ds(i * _L, _L)] = ga + gb
        return carry

    lax.fori_loop(0, _TBL // _L, tbl_body, jnp.int32(0), unroll=4)
    tc_cp.wait()

    out_acc = jnp.zeros((_L,), jnp.float32)
    for q in range(4):
        copies[q].wait()

        def row_body(r, out_acc, q=q):
            cv = jnp.zeros((_L,), jnp.float32)
            for i in range(_CVREGS):
                w = rows_v[q, r, pl.ds(i * _L, _L)]
                cv = cv + plsc.load_gather(tbl_v, [w & mask10])
                cv = cv + plsc.load_gather(tbl_v, [(w >> 10) & mask10])
                cv = cv + plsc.load_gather(tbl_v, [w >> 20])
            total_r = jnp.float32(_MANU_W / _N_SITES) * jnp.sum(cv)
            return jnp.where(lane_iota == (q % 2) * _QUART + r,
                             jnp.full((_L,), total_r), out_acc)

        out_acc = lax.fori_loop(0, _QUART, row_body, out_acc)
        if q % 2 == 1:
            out_v[pl.ds((q // 2) * _L, _L)] = \
                out_acc + tc_v[pl.ds((q // 2) * _L, _L)]
            out_acc = jnp.zeros((_L,), jnp.float32)

    pltpu.sync_copy(out_v, out_hbm.at[pl.ds(base, _ROWS_PER_W)])


@functools.partial(
    pl.kernel,
    mesh=plsc.VectorSubcoreMesh(core_axis_name="c", subcore_axis_name="s"),
    out_type=jax.ShapeDtypeStruct((_BATCH,), jnp.float32),
    scratch_types=[
        pltpu.VMEM((_COSTS_PAD,), jnp.float32),
        pltpu.VMEM((_TBL,), jnp.float32),
        pltpu.VMEM((4, _QUART, _CPAD), jnp.int32),
        pltpu.VMEM((_ROWS_PER_W,), jnp.float32),
        pltpu.VMEM((_ROWS_PER_W,), jnp.float32),
        pltpu.SemaphoreType.DMA,
        pltpu.SemaphoreType.DMA,
        pltpu.SemaphoreType.DMA,
        pltpu.SemaphoreType.DMA,
        pltpu.SemaphoreType.DMA,
    ],
    compiler_params=pltpu.CompilerParams(needs_layout_passes=False),
)
def _sc_costs(*refs):
    _sc_body(*refs)


def _tc_body(seq_ref, var_ref, cons_ref, out_ref, packed_ref):
    seq = seq_ref[...]
    p = seq[:, 0:_WORDS]
    for k in range(1, _PACK):
        p = p | (seq[:, k * _WORDS:(k + 1) * _WORDS] << (8 * k))

    pair = seq[:, 0:_N_SITES // 2] | (seq[:, _N_SITES // 2:] << 5)
    pair = jnp.concatenate(
        [pair, jnp.full((_TC_BLK, 3 * (_CWORDS) - _N_SITES // 2),
                        0x3FF, jnp.int32)], axis=1)
    cw = (pair[:, 0:_CWORDS]
          | (pair[:, _CWORDS:2 * _CWORDS] << 10)
          | (pair[:, 2 * _CWORDS:3 * _CWORDS] << 20))
    packed_ref[...] = jnp.concatenate(
        [cw, jnp.full((_TC_BLK, _CPAD - _CWORDS), 0x3FFFFFFF, jnp.int32)],
        axis=1)

    var = var_ref[...]
    pv = var[:, 0:_WORDS]
    for k in range(1, _PACK):
        pv = pv | (var[:, k * _WORDS:(k + 1) * _WORDS] << (8 * k))

    hi = jnp.int32(-2139062144)
    cov = jnp.zeros((_TC_BLK,), jnp.int32)
    covered = jnp.zeros((_TC_BLK,), jnp.int32)
    for v in range(_N_VARIANTS):
        x = p ^ pv[v][None, :]
        t = (hi - x) & hi
        bytes4 = lax.shift_right_logical(
            t, jnp.full(t.shape, 7, jnp.int32))
        t2 = bytes4 + (bytes4 >> 16)
        t3 = (t2 + (t2 >> 8)) & 0xFF
        cnt = jnp.sum(t3, axis=1)
        cov = cov + cnt
        covered = covered + (cnt > _N_SITES // 2).astype(jnp.int32)
    e_escape = jnp.float32(1.0) - cov.astype(jnp.float32) * \
        jnp.float32(1.0 / (_N_VARIANTS * _N_SITES))
    e_breadth = jnp.float32(1.0) - covered.astype(jnp.float32) * \
        jnp.float32(1.0 / _N_VARIANTS)

    cons_mean = jnp.sum(cons_ref[...]) * jnp.float32(1.0 / _N_SITES)
    z = (cons_mean * jnp.float32(1.0 / _N_STATES) - jnp.float32(_GL_CENTER)) \
        * jnp.float32(1.0 / _GL_WIDTH)
    e_auto = jnp.float32(1.0) - jnp.exp(jnp.float32(-0.5) * z * z)

    out_ref[...] = (jnp.float32(_ESCAPE_W) * e_escape
                    + jnp.float32(_BREADTH_W) * e_breadth
                    + jnp.float32(_AUTO_W) * e_auto)


_tc_matches = pl.pallas_call(
    _tc_body,
    grid=(_BATCH // _TC_BLK,),
    in_specs=[
        pl.BlockSpec((_TC_BLK, _N_SITES), lambda i: (i, 0)),
        pl.BlockSpec((_N_VARIANTS, _N_SITES), lambda i: (0, 0)),
        pl.BlockSpec((1, _N_SITES), lambda i: (0, 0)),
    ],
    out_specs=[
        pl.BlockSpec((_TC_BLK,), lambda i: (i,)),
        pl.BlockSpec((_TC_BLK, _CPAD), lambda i: (i, 0)),
    ],
    out_shape=[
        jax.ShapeDtypeStruct((_BATCH,), jnp.float32),
        jax.ShapeDtypeStruct((_BATCH, _CPAD), jnp.int32),
    ],
    compiler_params=pltpu.CompilerParams(
        dimension_semantics=("arbitrary",)),
)


def kernel(manufacturing_costs, conservation, sequence, variant_profiles):
    costs_pad = jnp.zeros((_COSTS_PAD,), jnp.float32)
    costs_pad = costs_pad.at[:_N_STATES].set(
        manufacturing_costs.astype(jnp.float32))
    tc_part, packed = _tc_matches(
        sequence, variant_profiles,
        conservation.astype(jnp.float32).reshape(1, _N_SITES))
    return _sc_costs(costs_pad, packed, tc_part)


# speedup vs baseline: 3.0476x; 1.0374x over previous
import functools

import jax
import jax.numpy as jnp
from jax import lax
from jax.experimental import pallas as pl
from jax.experimental.pallas import tpu as pltpu
from jax.experimental.pallas import tpu_sc as plsc

_N_SITES = 2048
_N_STATES = 21
_N_VARIANTS = 10
_BATCH = 1024
_ESCAPE_W, _MANU_W, _AUTO_W, _BREADTH_W = 1.0, 0.3, 0.5, 0.4
_GL_CENTER, _GL_WIDTH = 0.5, 0.15

_L = 16
_info = plsc.get_sparse_core_info()
_NC, _NS = _info.num_cores, _info.num_subcores
_NW = _NC * _NS
_ROWS_PER_W = _BATCH // _NW
_QUART = _ROWS_PER_W // 4
_PACK = 4
_WORDS = _N_SITES // _PACK
_PAIRS = 6
_CWORDS = _N_SITES // _PAIRS + 1
_CVREGS = 22
_CPAD = _CVREGS * _L
_TBL = 1024
_COSTS_PAD = 32

_TC_BLK = 128


def _sc_body(costs_hbm, packed_hbm, tc_hbm, out_hbm,
             costs_v, tbl_v, rows_v, tc_v, out_v,
             sem_a, sem_b, sem_c, sem_d, sem_e):
    wid = lax.axis_index("s") * _NC + lax.axis_index("c")
    base = wid * _ROWS_PER_W

    pltpu.sync_copy(costs_hbm, costs_v)
    tc_cp = pltpu.async_copy(tc_hbm.at[pl.ds(base, _ROWS_PER_W)],
                             tc_v, sem_e)
    sems = (sem_a, sem_b, sem_c, sem_d)
    copies = [
        pltpu.async_copy(packed_hbm.at[pl.ds(base + q * _QUART, _QUART)],
                         rows_v.at[q], sems[q])
        for q in range(4)
    ]

    lane_iota = lax.iota(jnp.int32, _L)
    mask10 = jnp.full((_L,), 0x3FF, jnp.int32)
    mask5 = jnp.full((_L,), 0x1F, jnp.int32)

    def tbl_body(i, carry):
        idx = lane_iota + i * _L
        ga = plsc.load_gather(costs_v, [idx & mask5])
        gb = plsc.load_gather(costs_v, [idx >> 5])
        tbl_v[pl.ds(i * _L, _L)] = ga + gb
        return carry

    lax.fori_loop(0, _TBL // _L, tbl_body, jnp.int32(0), unroll=4)
    tc_cp.wait()

    out_acc = jnp.zeros((_L,), jnp.float32)
    for q in range(4):
        copies[q].wait()

        def row_body(r, out_acc, q=q):
            cv = jnp.zeros((_L,), jnp.float32)
            for i in range(_CVREGS):
                w = rows_v[q, r, pl.ds(i * _L, _L)]
                cv = cv + plsc.load_gather(tbl_v, [w & mask10])
                cv = cv + plsc.load_gather(tbl_v, [(w >> 10) & mask10])
                cv = cv + plsc.load_gather(tbl_v, [w >> 20])
            total_r = jnp.float32(_MANU_W / _N_SITES) * jnp.sum(cv)
            return jnp.where(lane_iota == (q % 2) * _QUART + r,
                             jnp.full((_L,), total_r), out_acc)

        out_acc = lax.fori_loop(0, _QUART, row_body, out_acc)
        if q % 2 == 1:
            out_v[pl.ds((q // 2) * _L, _L)] = \
                out_acc + tc_v[pl.ds((q // 2) * _L, _L)]
            out_acc = jnp.zeros((_L,), jnp.float32)

    pltpu.sync_copy(out_v, out_hbm.at[pl.ds(base, _ROWS_PER_W)])


@functools.partial(
    pl.kernel,
    mesh=plsc.VectorSubcoreMesh(core_axis_name="c", subcore_axis_name="s"),
    out_type=jax.ShapeDtypeStruct((_BATCH,), jnp.float32),
    scratch_types=[
        pltpu.VMEM((_COSTS_PAD,), jnp.float32),
        pltpu.VMEM((_TBL,), jnp.float32),
        pltpu.VMEM((4, _QUART, _CPAD), jnp.int32),
        pltpu.VMEM((_ROWS_PER_W,), jnp.float32),
        pltpu.VMEM((_ROWS_PER_W,), jnp.float32),
        pltpu.SemaphoreType.DMA,
        pltpu.SemaphoreType.DMA,
        pltpu.SemaphoreType.DMA,
        pltpu.SemaphoreType.DMA,
        pltpu.SemaphoreType.DMA,
    ],
    compiler_params=pltpu.CompilerParams(needs_layout_passes=False),
)
def _sc_costs(*refs):
    _sc_body(*refs)


def _tc_body(seq_ref, var_ref, cons_ref, costs_ref, out_ref, packed_ref,
             costs_out_ref):
    costs_out_ref[...] = jnp.concatenate(
        [costs_ref[...], jnp.zeros((_COSTS_PAD - _N_STATES,), jnp.float32)])

    seq = seq_ref[...]
    p = seq[:, 0:_WORDS]
    for k in range(1, _PACK):
        p = p | (seq[:, k * _WORDS:(k + 1) * _WORDS] << (8 * k))

    pair = seq[:, 0:_N_SITES // 2] | (seq[:, _N_SITES // 2:] << 5)
    pair = jnp.concatenate(
        [pair, jnp.full((_TC_BLK, 3 * (_CWORDS) - _N_SITES // 2),
                        0x3FF, jnp.int32)], axis=1)
    cw = (pair[:, 0:_CWORDS]
          | (pair[:, _CWORDS:2 * _CWORDS] << 10)
          | (pair[:, 2 * _CWORDS:3 * _CWORDS] << 20))
    packed_ref[...] = jnp.concatenate(
        [cw, jnp.full((_TC_BLK, _CPAD - _CWORDS), 0x3FFFFFFF, jnp.int32)],
        axis=1)

    var = var_ref[...]
    pv = var[:, 0:_WORDS]
    for k in range(1, _PACK):
        pv = pv | (var[:, k * _WORDS:(k + 1) * _WORDS] << (8 * k))

    hi = jnp.int32(-2139062144)
    cov = jnp.zeros((_TC_BLK,), jnp.int32)
    covered = jnp.zeros((_TC_BLK,), jnp.int32)
    for v in range(_N_VARIANTS):
        x = p ^ pv[v][None, :]
        t = (hi - x) & hi
        bytes4 = lax.shift_right_logical(
            t, jnp.full(t.shape, 7, jnp.int32))
        t2 = bytes4 + (bytes4 >> 16)
        t3 = (t2 + (t2 >> 8)) & 0xFF
        cnt = jnp.sum(t3, axis=1)
        cov = cov + cnt
        covered = covered + (cnt > _N_SITES // 2).astype(jnp.int32)
    e_escape = jnp.float32(1.0) - cov.astype(jnp.float32) * \
        jnp.float32(1.0 / (_N_VARIANTS * _N_SITES))
    e_breadth = jnp.float32(1.0) - covered.astype(jnp.float32) * \
        jnp.float32(1.0 / _N_VARIANTS)

    cons_mean = jnp.sum(cons_ref[...]) * jnp.float32(1.0 / _N_SITES)
    z = (cons_mean * jnp.float32(1.0 / _N_STATES) - jnp.float32(_GL_CENTER)) \
        * jnp.float32(1.0 / _GL_WIDTH)
    e_auto = jnp.float32(1.0) - jnp.exp(jnp.float32(-0.5) * z * z)

    out_ref[...] = (jnp.float32(_ESCAPE_W) * e_escape
                    + jnp.float32(_BREADTH_W) * e_breadth
                    + jnp.float32(_AUTO_W) * e_auto)


_tc_matches = pl.pallas_call(
    _tc_body,
    grid=(_BATCH // _TC_BLK,),
    in_specs=[
        pl.BlockSpec((_TC_BLK, _N_SITES), lambda i: (i, 0)),
        pl.BlockSpec((_N_VARIANTS, _N_SITES), lambda i: (0, 0)),
        pl.BlockSpec((1, _N_SITES), lambda i: (0, 0)),
        pl.BlockSpec((_N_STATES,), lambda i: (0,)),
    ],
    out_specs=[
        pl.BlockSpec((_TC_BLK,), lambda i: (i,)),
        pl.BlockSpec((_TC_BLK, _CPAD), lambda i: (i, 0)),
        pl.BlockSpec((_COSTS_PAD,), lambda i: (0,)),
    ],
    out_shape=[
        jax.ShapeDtypeStruct((_BATCH,), jnp.float32),
        jax.ShapeDtypeStruct((_BATCH, _CPAD), jnp.int32),
        jax.ShapeDtypeStruct((_COSTS_PAD,), jnp.float32),
    ],
    compiler_params=pltpu.CompilerParams(
        dimension_semantics=("arbitrary",)),
)


def kernel(manufacturing_costs, conservation, sequence, variant_profiles):
    tc_part, packed, costs_pad = _tc_matches(
        sequence, variant_profiles,
        conservation.astype(jnp.float32).reshape(1, _N_SITES),
        manufacturing_costs.astype(jnp.float32))
    return _sc_costs(costs_pad, packed, tc_part)


# speedup vs baseline: 3.1826x; 1.0443x over previous
import functools

import jax
import jax.numpy as jnp
from jax import lax
from jax.experimental import pallas as pl
from jax.experimental.pallas import tpu as pltpu
from jax.experimental.pallas import tpu_sc as plsc

_N_SITES = 2048
_N_STATES = 21
_N_VARIANTS = 10
_BATCH = 1024
_ESCAPE_W, _MANU_W, _AUTO_W, _BREADTH_W = 1.0, 0.3, 0.5, 0.4
_GL_CENTER, _GL_WIDTH = 0.5, 0.15

_L = 16
_info = plsc.get_sparse_core_info()
_NC, _NS = _info.num_cores, _info.num_subcores
_NW = _NC * _NS
_ROWS_PER_W = _BATCH // _NW
_QUART = _ROWS_PER_W // 4
_PACK = 4
_WORDS = _N_SITES // _PACK
_PAIRS = 6
_CWORDS = _N_SITES // _PAIRS + 1
_CVREGS = 22
_CPAD = _CVREGS * _L
_TBL = 1024
_COSTS_PAD = 32

_TC_BLK = 128


def _sc_body(costs_hbm, packed_hbm, tc_hbm, out_hbm,
             costs_v, tbl_v, rows_v, tc_v, out_v,
             sem_a, sem_b, sem_c, sem_d, sem_e):
    wid = lax.axis_index("s") * _NC + lax.axis_index("c")
    base = wid * _ROWS_PER_W

    pltpu.sync_copy(costs_hbm, costs_v)
    tc_cp = pltpu.async_copy(tc_hbm.at[pl.ds(base, _ROWS_PER_W)],
                             tc_v, sem_e)
    sems = (sem_a, sem_b, sem_c, sem_d)
    copies = [
        pltpu.async_copy(packed_hbm.at[pl.ds(base + q * _QUART, _QUART)],
                         rows_v.at[q], sems[q])
        for q in range(4)
    ]

    lane_iota = lax.iota(jnp.int32, _L)
    mask10 = jnp.full((_L,), 0x3FF, jnp.int32)
    mask5 = jnp.full((_L,), 0x1F, jnp.int32)

    def tbl_body(i, carry):
        idx = lane_iota + i * _L
        ga = plsc.load_gather(costs_v, [idx & mask5])
        gb = plsc.load_gather(costs_v, [idx >> 5])
        tbl_v[pl.ds(i * _L, _L)] = ga + gb
        return carry

    lax.fori_loop(0, _TBL // _L, tbl_body, jnp.int32(0), unroll=4)
    tc_cp.wait()

    out_acc = jnp.zeros((_L,), jnp.float32)
    for q in range(4):
        copies[q].wait()

        def row_body(r, out_acc, q=q):
            cv = jnp.zeros((_L,), jnp.float32)
            for i in range(_CVREGS):
                w = rows_v[q, r, pl.ds(i * _L, _L)]
                cv = cv + plsc.load_gather(tbl_v, [w & mask10])
                cv = cv + plsc.load_gather(tbl_v, [(w >> 10) & mask10])
                cv = cv + plsc.load_gather(tbl_v, [w >> 20])
            total_r = jnp.float32(_MANU_W / _N_SITES) * jnp.sum(cv)
            return jnp.where(lane_iota == (q % 2) * _QUART + r,
                             jnp.full((_L,), total_r), out_acc)

        out_acc = lax.fori_loop(0, _QUART, row_body, out_acc)
        if q % 2 == 1:
            out_v[pl.ds((q // 2) * _L, _L)] = \
                out_acc + tc_v[pl.ds((q // 2) * _L, _L)]
            out_acc = jnp.zeros((_L,), jnp.float32)

    pltpu.sync_copy(out_v, out_hbm.at[pl.ds(base, _ROWS_PER_W)])


@functools.partial(
    pl.kernel,
    mesh=plsc.VectorSubcoreMesh(core_axis_name="c", subcore_axis_name="s"),
    out_type=jax.ShapeDtypeStruct((_BATCH,), jnp.float32),
    scratch_types=[
        pltpu.VMEM((_COSTS_PAD,), jnp.float32),
        pltpu.VMEM((_TBL,), jnp.float32),
        pltpu.VMEM((4, _QUART, _CPAD), jnp.int32),
        pltpu.VMEM((_ROWS_PER_W,), jnp.float32),
        pltpu.VMEM((_ROWS_PER_W,), jnp.float32),
        pltpu.SemaphoreType.DMA,
        pltpu.SemaphoreType.DMA,
        pltpu.SemaphoreType.DMA,
        pltpu.SemaphoreType.DMA,
        pltpu.SemaphoreType.DMA,
    ],
    compiler_params=pltpu.CompilerParams(needs_layout_passes=False),
)
def _sc_costs(*refs):
    _sc_body(*refs)


def _tc_body(seq_ref, var_ref, cons_ref, costs_ref, out_ref, packed_ref,
             costs_out_ref):
    costs_out_ref[...] = jnp.concatenate(
        [costs_ref[...], jnp.zeros((_COSTS_PAD - _N_STATES,), jnp.float32)])

    seq = seq_ref[...]
    p = seq[:, 0:_WORDS]
    for k in range(1, _PACK):
        p = p | (seq[:, k * _WORDS:(k + 1) * _WORDS] << (8 * k))

    pair = seq[:, 0:_N_SITES // 2] | (seq[:, _N_SITES // 2:] << 5)
    pair = jnp.concatenate(
        [pair, jnp.full((_TC_BLK, 3 * (_CWORDS) - _N_SITES // 2),
                        0x3FF, jnp.int32)], axis=1)
    cw = (pair[:, 0:_CWORDS]
          | (pair[:, _CWORDS:2 * _CWORDS] << 10)
          | (pair[:, 2 * _CWORDS:3 * _CWORDS] << 20))
    packed_ref[...] = jnp.concatenate(
        [cw, jnp.full((_TC_BLK, _CPAD - _CWORDS), 0x3FFFFFFF, jnp.int32)],
        axis=1)

    var = var_ref[...]
    pv = var[:, 0:_WORDS]
    for k in range(1, _PACK):
        pv = pv | (var[:, k * _WORDS:(k + 1) * _WORDS] << (8 * k))

    hi = jnp.int32(-2139062144)
    cov = jnp.zeros((_TC_BLK,), jnp.int32)
    covered = jnp.zeros((_TC_BLK,), jnp.int32)
    for v in range(_N_VARIANTS):
        acc = jnp.zeros((_TC_BLK, _TC_BLK), jnp.int32)
        for wc in range(_WORDS // _TC_BLK):
            x = p[:, wc * _TC_BLK:(wc + 1) * _TC_BLK] \
                ^ pv[v][None, wc * _TC_BLK:(wc + 1) * _TC_BLK]
            t = (hi - x) & hi
            acc = acc + lax.shift_right_logical(
                t, jnp.full(t.shape, 7, jnp.int32))
        t2 = acc + (acc >> 16)
        t3 = (t2 + (t2 >> 8)) & 0xFF
        cnt = jnp.sum(t3, axis=1)
        cov = cov + cnt
        covered = covered + (cnt > _N_SITES // 2).astype(jnp.int32)
    e_escape = jnp.float32(1.0) - cov.astype(jnp.float32) * \
        jnp.float32(1.0 / (_N_VARIANTS * _N_SITES))
    e_breadth = jnp.float32(1.0) - covered.astype(jnp.float32) * \
        jnp.float32(1.0 / _N_VARIANTS)

    cons_mean = jnp.sum(cons_ref[...]) * jnp.float32(1.0 / _N_SITES)
    z = (cons_mean * jnp.float32(1.0 / _N_STATES) - jnp.float32(_GL_CENTER)) \
        * jnp.float32(1.0 / _GL_WIDTH)
    e_auto = jnp.float32(1.0) - jnp.exp(jnp.float32(-0.5) * z * z)

    out_ref[...] = (jnp.float32(_ESCAPE_W) * e_escape
                    + jnp.float32(_BREADTH_W) * e_breadth
                    + jnp.float32(_AUTO_W) * e_auto)


_tc_matches = pl.pallas_call(
    _tc_body,
    grid=(_BATCH // _TC_BLK,),
    in_specs=[
        pl.BlockSpec((_TC_BLK, _N_SITES), lambda i: (i, 0)),
        pl.BlockSpec((_N_VARIANTS, _N_SITES), lambda i: (0, 0)),
        pl.BlockSpec((1, _N_SITES), lambda i: (0, 0)),
        pl.BlockSpec((_N_STATES,), lambda i: (0,)),
    ],
    out_specs=[
        pl.BlockSpec((_TC_BLK,), lambda i: (i,)),
        pl.BlockSpec((_TC_BLK, _CPAD), lambda i: (i, 0)),
        pl.BlockSpec((_COSTS_PAD,), lambda i: (0,)),
    ],
    out_shape=[
        jax.ShapeDtypeStruct((_BATCH,), jnp.float32),
        jax.ShapeDtypeStruct((_BATCH, _CPAD), jnp.int32),
        jax.ShapeDtypeStruct((_COSTS_PAD,), jnp.float32),
    ],
    compiler_params=pltpu.CompilerParams(
        dimension_semantics=("arbitrary",)),
)


def kernel(manufacturing_costs, conservation, sequence, variant_profiles):
    tc_part, packed, costs_pad = _tc_matches(
        sequence, variant_profiles,
        conservation.astype(jnp.float32).reshape(1, _N_SITES),
        manufacturing_costs.astype(jnp.float32))
    return _sc_costs(costs_pad, packed, tc_part)


# speedup vs baseline: 3.2811x; 1.0309x over previous
import functools

import jax
import jax.numpy as jnp
from jax import lax
from jax.experimental import pallas as pl
from jax.experimental.pallas import tpu as pltpu
from jax.experimental.pallas import tpu_sc as plsc

_N_SITES = 2048
_N_STATES = 21
_N_VARIANTS = 10
_BATCH = 1024
_ESCAPE_W, _MANU_W, _AUTO_W, _BREADTH_W = 1.0, 0.3, 0.5, 0.4
_GL_CENTER, _GL_WIDTH = 0.5, 0.15

_L = 16
_info = plsc.get_sparse_core_info()
_NC, _NS = _info.num_cores, _info.num_subcores
_NW = _NC * _NS
_ROWS_PER_W = _BATCH // _NW
_QUART = _ROWS_PER_W // 4
_PACK = 4
_WORDS = _N_SITES // _PACK
_PAIRS = 6
_CWORDS = _N_SITES // _PAIRS + 1
_CVREGS = 22
_CPAD = _CVREGS * _L
_TBL = 1024
_COSTS_PAD = 32

_TC_BLK = 256
_ACC_W = 128


def _sc_body(costs_hbm, packed_hbm, tc_hbm, out_hbm,
             costs_v, tbl_v, rows_v, tc_v, out_v,
             sem_a, sem_b, sem_c, sem_d, sem_e):
    wid = lax.axis_index("s") * _NC + lax.axis_index("c")
    base = wid * _ROWS_PER_W

    pltpu.sync_copy(costs_hbm, costs_v)
    tc_cp = pltpu.async_copy(tc_hbm.at[pl.ds(base, _ROWS_PER_W)],
                             tc_v, sem_e)
    sems = (sem_a, sem_b, sem_c, sem_d)
    copies = [
        pltpu.async_copy(packed_hbm.at[pl.ds(base + q * _QUART, _QUART)],
                         rows_v.at[q], sems[q])
        for q in range(4)
    ]

    lane_iota = lax.iota(jnp.int32, _L)
    mask10 = jnp.full((_L,), 0x3FF, jnp.int32)
    mask5 = jnp.full((_L,), 0x1F, jnp.int32)

    def tbl_body(i, carry):
        idx = lane_iota + i * _L
        ga = plsc.load_gather(costs_v, [idx & mask5])
        gb = plsc.load_gather(costs_v, [idx >> 5])
        tbl_v[pl.ds(i * _L, _L)] = ga + gb
        return carry

    lax.fori_loop(0, _TBL // _L, tbl_body, jnp.int32(0), unroll=4)
    tc_cp.wait()

    out_acc = jnp.zeros((_L,), jnp.float32)
    for q in range(4):
        copies[q].wait()

        def row_body(r, out_acc, q=q):
            cv = jnp.zeros((_L,), jnp.float32)
            for i in range(_CVREGS):
                w = rows_v[q, r, pl.ds(i * _L, _L)]
                cv = cv + plsc.load_gather(tbl_v, [w & mask10])
                cv = cv + plsc.load_gather(tbl_v, [(w >> 10) & mask10])
                cv = cv + plsc.load_gather(tbl_v, [w >> 20])
            total_r = jnp.float32(_MANU_W / _N_SITES) * jnp.sum(cv)
            return jnp.where(lane_iota == (q % 2) * _QUART + r,
                             jnp.full((_L,), total_r), out_acc)

        out_acc = lax.fori_loop(0, _QUART, row_body, out_acc)
        if q % 2 == 1:
            out_v[pl.ds((q // 2) * _L, _L)] = \
                out_acc + tc_v[pl.ds((q // 2) * _L, _L)]
            out_acc = jnp.zeros((_L,), jnp.float32)

    pltpu.sync_copy(out_v, out_hbm.at[pl.ds(base, _ROWS_PER_W)])


@functools.partial(
    pl.kernel,
    mesh=plsc.VectorSubcoreMesh(core_axis_name="c", subcore_axis_name="s"),
    out_type=jax.ShapeDtypeStruct((_BATCH,), jnp.float32),
    scratch_types=[
        pltpu.VMEM((_COSTS_PAD,), jnp.float32),
        pltpu.VMEM((_TBL,), jnp.float32),
        pltpu.VMEM((4, _QUART, _CPAD), jnp.int32),
        pltpu.VMEM((_ROWS_PER_W,), jnp.float32),
        pltpu.VMEM((_ROWS_PER_W,), jnp.float32),
        pltpu.SemaphoreType.DMA,
        pltpu.SemaphoreType.DMA,
        pltpu.SemaphoreType.DMA,
        pltpu.SemaphoreType.DMA,
        pltpu.SemaphoreType.DMA,
    ],
    compiler_params=pltpu.CompilerParams(needs_layout_passes=False),
)
def _sc_costs(*refs):
    _sc_body(*refs)


def _tc_body(seq_ref, var_ref, cons_ref, costs_ref, out_ref, packed_ref,
             costs_out_ref):
    costs_out_ref[...] = jnp.concatenate(
        [costs_ref[...], jnp.zeros((_COSTS_PAD - _N_STATES,), jnp.float32)])

    seq = seq_ref[...]
    p = seq[:, 0:_WORDS]
    for k in range(1, _PACK):
        p = p | (seq[:, k * _WORDS:(k + 1) * _WORDS] << (8 * k))

    pair = seq[:, 0:_N_SITES // 2] | (seq[:, _N_SITES // 2:] << 5)
    pair = jnp.concatenate(
        [pair, jnp.full((_TC_BLK, 3 * (_CWORDS) - _N_SITES // 2),
                        0x3FF, jnp.int32)], axis=1)
    cw = (pair[:, 0:_CWORDS]
          | (pair[:, _CWORDS:2 * _CWORDS] << 10)
          | (pair[:, 2 * _CWORDS:3 * _CWORDS] << 20))
    packed_ref[...] = jnp.concatenate(
        [cw, jnp.full((_TC_BLK, _CPAD - _CWORDS), 0x3FFFFFFF, jnp.int32)],
        axis=1)

    var = var_ref[...]
    pv = var[:, 0:_WORDS]
    for k in range(1, _PACK):
        pv = pv | (var[:, k * _WORDS:(k + 1) * _WORDS] << (8 * k))

    hi = jnp.int32(-2139062144)
    cov = jnp.zeros((_TC_BLK,), jnp.int32)
    covered = jnp.zeros((_TC_BLK,), jnp.int32)
    for v in range(_N_VARIANTS):
        acc = jnp.zeros((_TC_BLK, _ACC_W), jnp.int32)
        for wc in range(_WORDS // _ACC_W):
            x = p[:, wc * _ACC_W:(wc + 1) * _ACC_W] \
                ^ pv[v][None, wc * _ACC_W:(wc + 1) * _ACC_W]
            t = (hi - x) & hi
            acc = acc + lax.shift_right_logical(
                t, jnp.full(t.shape, 7, jnp.int32))
        t2 = acc + (acc >> 16)
        t3 = (t2 + (t2 >> 8)) & 0xFF
        cnt = jnp.sum(t3, axis=1)
        cov = cov + cnt
        covered = covered + (cnt > _N_SITES // 2).astype(jnp.int32)
    e_escape = jnp.float32(1.0) - cov.astype(jnp.float32) * \
        jnp.float32(1.0 / (_N_VARIANTS * _N_SITES))
    e_breadth = jnp.float32(1.0) - covered.astype(jnp.float32) * \
        jnp.float32(1.0 / _N_VARIANTS)

    cons_mean = jnp.sum(cons_ref[...]) * jnp.float32(1.0 / _N_SITES)
    z = (cons_mean * jnp.float32(1.0 / _N_STATES) - jnp.float32(_GL_CENTER)) \
        * jnp.float32(1.0 / _GL_WIDTH)
    e_auto = jnp.float32(1.0) - jnp.exp(jnp.float32(-0.5) * z * z)

    out_ref[...] = (jnp.float32(_ESCAPE_W) * e_escape
                    + jnp.float32(_BREADTH_W) * e_breadth
                    + jnp.float32(_AUTO_W) * e_auto)


_tc_matches = pl.pallas_call(
    _tc_body,
    grid=(_BATCH // _TC_BLK,),
    in_specs=[
        pl.BlockSpec((_TC_BLK, _N_SITES), lambda i: (i, 0)),
        pl.BlockSpec((_N_VARIANTS, _N_SITES), lambda i: (0, 0)),
        pl.BlockSpec((_N_SITES,), lambda i: (0,)),
        pl.BlockSpec((_N_STATES,), lambda i: (0,)),
    ],
    out_specs=[
        pl.BlockSpec((_TC_BLK,), lambda i: (i,)),
        pl.BlockSpec((_TC_BLK, _CPAD), lambda i: (i, 0)),
        pl.BlockSpec((_COSTS_PAD,), lambda i: (0,)),
    ],
    out_shape=[
        jax.ShapeDtypeStruct((_BATCH,), jnp.float32),
        jax.ShapeDtypeStruct((_BATCH, _CPAD), jnp.int32),
        jax.ShapeDtypeStruct((_COSTS_PAD,), jnp.float32),
    ],
    compiler_params=pltpu.CompilerParams(
        dimension_semantics=("arbitrary",)),
)


def kernel(manufacturing_costs, conservation, sequence, variant_profiles):
    tc_part, packed, costs_pad = _tc_matches(
        sequence, variant_profiles, conservation.astype(jnp.float32),
        manufacturing_costs.astype(jnp.float32))
    return _sc_costs(costs_pad, packed, tc_part)
